# baseline (speedup 1.0000x reference)
import jax
import jax.numpy as jnp
from jax.experimental import pallas as pl
from jax.experimental.pallas import tpu as pltpu

_B = 8
_TOTAL = 8192
_D = 512
_H = _D // 2
_EPS = 1e-6
_T = 2048
_NC = 8
_C = _T // _NC


def _rms(y, ln):
    var = jnp.mean(y * y, axis=-1, keepdims=True)
    return y * jax.lax.rsqrt(var + _EPS) * ln


def _chunk(x, ca1, ca2, w1_ref, w2_ref, b1, b2, ln):
    row = jax.lax.broadcasted_iota(jnp.int32, (_C, 1), 0)
    c1 = jnp.dot(x, w1_ref[:], preferred_element_type=jnp.float32)
    c1h = c1[:, :_H]
    o1 = (jnp.where(row == 0, ca1, pltpu.roll(c1h, 1, axis=0))
          + c1[:, _H:] + b1)
    c2 = jnp.dot(o1, w2_ref[:], preferred_element_type=jnp.float32)
    c2d = c2[:, :_D]
    y = (jnp.where(row == 0, ca2, pltpu.roll(c2d, 1, axis=0))
         + c2[:, _D:] + b2 + x)
    return _rms(y, ln), c1h[_C - 1:_C, :], c2d[_C - 1:_C, :]


def _dense_kernel(x_ref, w1_ref, w2_ref, b1_ref, b2_ref, ln_ref, out_ref,
                  carry1, carry2):
    i = pl.program_id(0)

    @pl.when(i == 0)
    def _init():
        carry1[:] = jnp.zeros_like(carry1)
        carry2[:] = jnp.zeros_like(carry2)

    x = x_ref[:]
    ca1 = carry1[:]
    ca2 = carry2[:]
    for k in range(_NC):
        xk = x[k * _C:(k + 1) * _C, :]
        outk, ca1, ca2 = _chunk(xk, ca1, ca2, w1_ref, w2_ref,
                                b1_ref[:], b2_ref[:], ln_ref[:])
        out_ref[k * _C:(k + 1) * _C, :] = outk
    carry1[:] = ca1
    carry2[:] = ca2


def _fixup_kernel(s_ref, x_hbm, out_in_hbm, lf1_ref, lf2_ref, w1_ref,
                  w2_ref, b1_ref, b2_ref, ln_ref, out_hbm, lf1n_ref,
                  lf2n_ref, xr_s, yo_s, in_sems, out_sems):
    f32 = jnp.float32
    del out_in_hbm

    def clamp(v):
        return jnp.minimum(jnp.maximum(v, 0), _TOTAL - 1)

    for b in range(_B):
        r = s_ref[b]
        last = s_ref[b + 1] - 1
        for j, idx in enumerate((r, r + 1, last, last - 1)):
            pltpu.make_async_copy(
                x_hbm.at[pl.ds(clamp(idx), 1), :],
                xr_s.at[pl.ds(4 * b + j, 1), :],
                in_sems.at[4 * b + j],
            ).start()
    for b in range(_B):
        for j in range(4):
            pltpu.make_async_copy(
                x_hbm.at[pl.ds(0, 1), :],
                xr_s.at[pl.ds(4 * b + j, 1), :],
                in_sems.at[4 * b + j],
            ).wait()

    X = xr_s[:]
    C1 = jnp.dot(X, w1_ref[:], preferred_element_type=f32)
    cproj1 = jnp.dot(lf1_ref[:], w1_ref[:, :_H],
                     preferred_element_type=f32)
    cproj2 = jnp.dot(lf2_ref[:], w2_ref[:, :_D],
                     preferred_element_type=f32)

    b1 = b1_ref[:]
    b2 = b2_ref[:]
    ln = ln_ref[:]

    o1_rows = []
    for b in range(_B):
        o1_start = cproj1[b:b + 1, :] + C1[4 * b:4 * b + 1, _H:] + b1
        o1_succ = (C1[4 * b:4 * b + 1, :_H]
                   + C1[4 * b + 1:4 * b + 2, _H:] + b1)
        o1_rows.append(o1_start)
        o1_rows.append(o1_succ)
    O1 = jnp.concatenate(o1_rows, axis=0)
    C2 = jnp.dot(O1, w2_ref[:], preferred_element_type=f32)

    for b in range(_B):
        r = s_ref[b]
        lens_b = s_ref[b + 1] - s_ref[b]
        xr = X[4 * b:4 * b + 1, :]
        xr1 = X[4 * b + 1:4 * b + 2, :]
        y_r = (cproj2[b:b + 1, :] + C2[2 * b:2 * b + 1, _D:] + b2 + xr)
        y_r1 = (C2[2 * b:2 * b + 1, :_D]
                + C2[2 * b + 1:2 * b + 2, _D:] + b2 + xr1)
        yo_s[2 * b:2 * b + 1, :] = _rms(y_r, ln)
        yo_s[2 * b + 1:2 * b + 2, :] = _rms(y_r1, ln)

        o1_start = O1[2 * b:2 * b + 1, :]
        o1_last_dense = (C1[4 * b + 3:4 * b + 4, :_H]
                         + C1[4 * b + 2:4 * b + 3, _H:] + b1)
        o1_last = jnp.where(lens_b == 1, o1_start, o1_last_dense)
        lf1n_ref[b:b + 1, :] = jnp.where(lens_b > 0,
                                         X[4 * b + 2:4 * b + 3, :],
                                         lf1_ref[b:b + 1, :])
        lf2n_ref[b:b + 1, :] = jnp.where(lens_b > 0, o1_last,
                                         lf2_ref[b:b + 1, :])

        @pl.when(lens_b > 0)
        def _(b=b, r=r):
            pltpu.make_async_copy(
                yo_s.at[pl.ds(2 * b, 1), :],
                out_hbm.at[pl.ds(r, 1), :],
                out_sems.at[2 * b],
            ).start()

        @pl.when(lens_b > 1)
        def _(b=b, r=r):
            pltpu.make_async_copy(
                yo_s.at[pl.ds(2 * b + 1, 1), :],
                out_hbm.at[pl.ds(r + 1, 1), :],
                out_sems.at[2 * b + 1],
            ).start()

    for b in range(_B):
        lens_b = s_ref[b + 1] - s_ref[b]

        @pl.when(lens_b > 0)
        def _(b=b):
            pltpu.make_async_copy(
                yo_s.at[pl.ds(2 * b, 1), :],
                out_hbm.at[pl.ds(0, 1), :],
                out_sems.at[2 * b],
            ).wait()

        @pl.when(lens_b > 1)
        def _(b=b):
            pltpu.make_async_copy(
                yo_s.at[pl.ds(2 * b + 1, 1), :],
                out_hbm.at[pl.ds(0, 1), :],
                out_sems.at[2 * b + 1],
            ).wait()


@jax.jit
def kernel(inputs, lf1_cache, lf2_cache, conv1_weight, conv2_weight,
           conv1_bias, conv2_bias, ln_weight, cu_seqlens):
    lf1 = lf1_cache.reshape(_B, _D)
    lf2 = lf2_cache.reshape(_B, _H)
    b1 = conv1_bias.reshape(1, _H)
    b2 = conv2_bias.reshape(1, _D)
    ln = ln_weight.reshape(1, _D)
    n_tiles = _TOTAL // _T

    dense_out = pl.pallas_call(
        _dense_kernel,
        grid=(n_tiles,),
        in_specs=[
            pl.BlockSpec((_T, _D), lambda i: (i, 0)),
            pl.BlockSpec((_D, _D), lambda i: (0, 0)),
            pl.BlockSpec((_H, 2 * _D), lambda i: (0, 0)),
            pl.BlockSpec((1, _H), lambda i: (0, 0)),
            pl.BlockSpec((1, _D), lambda i: (0, 0)),
            pl.BlockSpec((1, _D), lambda i: (0, 0)),
        ],
        out_specs=pl.BlockSpec((_T, _D), lambda i: (i, 0)),
        out_shape=jax.ShapeDtypeStruct((_TOTAL, _D), jnp.float32),
        scratch_shapes=[
            pltpu.VMEM((1, _H), jnp.float32),
            pltpu.VMEM((1, _D), jnp.float32),
        ],
        compiler_params=pltpu.CompilerParams(
            dimension_semantics=("arbitrary",)),
    )(inputs, conv1_weight, conv2_weight, b1, b2, ln)

    grid_spec = pltpu.PrefetchScalarGridSpec(
        num_scalar_prefetch=1,
        grid=(1,),
        in_specs=[
            pl.BlockSpec(memory_space=pl.ANY),
            pl.BlockSpec(memory_space=pl.ANY),
            pl.BlockSpec((_B, _D), lambda i, s: (0, 0)),
            pl.BlockSpec((_B, _H), lambda i, s: (0, 0)),
            pl.BlockSpec((_D, _D), lambda i, s: (0, 0)),
            pl.BlockSpec((_H, 2 * _D), lambda i, s: (0, 0)),
            pl.BlockSpec((1, _H), lambda i, s: (0, 0)),
            pl.BlockSpec((1, _D), lambda i, s: (0, 0)),
            pl.BlockSpec((1, _D), lambda i, s: (0, 0)),
        ],
        out_specs=[
            pl.BlockSpec(memory_space=pl.ANY),
            pl.BlockSpec((_B, _D), lambda i, s: (0, 0)),
            pl.BlockSpec((_B, _H), lambda i, s: (0, 0)),
        ],
        scratch_shapes=[
            pltpu.VMEM((4 * _B, _D), jnp.float32),
            pltpu.VMEM((2 * _B, _D), jnp.float32),
            pltpu.SemaphoreType.DMA((4 * _B,)),
            pltpu.SemaphoreType.DMA((2 * _B,)),
        ],
    )

    out, lf1n, lf2n = pl.pallas_call(
        _fixup_kernel,
        grid_spec=grid_spec,
        out_shape=[
            jax.ShapeDtypeStruct((_TOTAL, _D), jnp.float32),
            jax.ShapeDtypeStruct((_B, _D), jnp.float32),
            jax.ShapeDtypeStruct((_B, _H), jnp.float32),
        ],
        input_output_aliases={2: 0},
    )(cu_seqlens, inputs, dense_out, lf1, lf2, conv1_weight, conv2_weight,
      b1, b2, ln)

    return out, lf1n.reshape(_B, 1, _D), lf2n.reshape(_B, 1, _H)


# speedup vs baseline: 1.0798x; 1.0798x over previous
import jax
import jax.numpy as jnp
from jax.experimental import pallas as pl
from jax.experimental.pallas import tpu as pltpu

_B = 8
_TOTAL = 8192
_D = 512
_H = _D // 2
_EPS = 1e-6
_T = 2048
_NC = 4
_C = _T // _NC


def _rms(y, ln):
    var = jnp.mean(y * y, axis=-1, keepdims=True)
    return y * jax.lax.rsqrt(var + _EPS) * ln


def _chunk(x, ca1, ca2, w1_ref, w2_ref, b1, b2, ln):
    row = jax.lax.broadcasted_iota(jnp.int32, (_C, 1), 0)
    c1 = jnp.dot(x, w1_ref[:], preferred_element_type=jnp.float32)
    c1h = c1[:, :_H]
    o1 = (jnp.where(row == 0, ca1, pltpu.roll(c1h, 1, axis=0))
          + c1[:, _H:] + b1)
    c2 = jnp.dot(o1, w2_ref[:], preferred_element_type=jnp.float32)
    c2d = c2[:, :_D]
    y = (jnp.where(row == 0, ca2, pltpu.roll(c2d, 1, axis=0))
         + c2[:, _D:] + b2 + x)
    return _rms(y, ln), c1h[_C - 1:_C, :], c2d[_C - 1:_C, :]


def _dense_kernel(x_ref, w1_ref, w2_ref, b1_ref, b2_ref, ln_ref, out_ref,
                  carry1, carry2):
    i = pl.program_id(0)

    @pl.when(i == 0)
    def _init():
        carry1[:] = jnp.zeros_like(carry1)
        carry2[:] = jnp.zeros_like(carry2)

    x = x_ref[:]
    ca1 = carry1[:]
    ca2 = carry2[:]
    for k in range(_NC):
        xk = x[k * _C:(k + 1) * _C, :]
        outk, ca1, ca2 = _chunk(xk, ca1, ca2, w1_ref, w2_ref,
                                b1_ref[:], b2_ref[:], ln_ref[:])
        out_ref[k * _C:(k + 1) * _C, :] = outk
    carry1[:] = ca1
    carry2[:] = ca2


def _fixup_kernel(s_ref, x_hbm, out_in_hbm, lf1_ref, lf2_ref, w1_ref,
                  w2_ref, b1_ref, b2_ref, ln_ref, out_hbm, lf1n_ref,
                  lf2n_ref, xr_s, yo_s, in_sems, out_sems):
    f32 = jnp.float32
    del out_in_hbm

    def clamp(v):
        return jnp.minimum(jnp.maximum(v, 0), _TOTAL - 1)

    for b in range(_B):
        r = s_ref[b]
        last = s_ref[b + 1] - 1
        for j, idx in enumerate((r, r + 1, last, last - 1)):
            pltpu.make_async_copy(
                x_hbm.at[pl.ds(clamp(idx), 1), :],
                xr_s.at[pl.ds(4 * b + j, 1), :],
                in_sems.at[4 * b + j],
            ).start()
    for b in range(_B):
        for j in range(4):
            pltpu.make_async_copy(
                x_hbm.at[pl.ds(0, 1), :],
                xr_s.at[pl.ds(4 * b + j, 1), :],
                in_sems.at[4 * b + j],
            ).wait()

    X = xr_s[:]
    C1 = jnp.dot(X, w1_ref[:], preferred_element_type=f32)
    cproj1 = jnp.dot(lf1_ref[:], w1_ref[:, :_H],
                     preferred_element_type=f32)
    cproj2 = jnp.dot(lf2_ref[:], w2_ref[:, :_D],
                     preferred_element_type=f32)

    b1 = b1_ref[:]
    b2 = b2_ref[:]
    ln = ln_ref[:]

    o1_rows = []
    for b in range(_B):
        o1_start = cproj1[b:b + 1, :] + C1[4 * b:4 * b + 1, _H:] + b1
        o1_succ = (C1[4 * b:4 * b + 1, :_H]
                   + C1[4 * b + 1:4 * b + 2, _H:] + b1)
        o1_rows.append(o1_start)
        o1_rows.append(o1_succ)
    O1 = jnp.concatenate(o1_rows, axis=0)
    C2 = jnp.dot(O1, w2_ref[:], preferred_element_type=f32)

    for b in range(_B):
        r = s_ref[b]
        lens_b = s_ref[b + 1] - s_ref[b]
        xr = X[4 * b:4 * b + 1, :]
        xr1 = X[4 * b + 1:4 * b + 2, :]
        y_r = (cproj2[b:b + 1, :] + C2[2 * b:2 * b + 1, _D:] + b2 + xr)
        y_r1 = (C2[2 * b:2 * b + 1, :_D]
                + C2[2 * b + 1:2 * b + 2, _D:] + b2 + xr1)
        yo_s[2 * b:2 * b + 1, :] = _rms(y_r, ln)
        yo_s[2 * b + 1:2 * b + 2, :] = _rms(y_r1, ln)

        o1_start = O1[2 * b:2 * b + 1, :]
        o1_last_dense = (C1[4 * b + 3:4 * b + 4, :_H]
                         + C1[4 * b + 2:4 * b + 3, _H:] + b1)
        o1_last = jnp.where(lens_b == 1, o1_start, o1_last_dense)
        lf1n_ref[b:b + 1, :] = jnp.where(lens_b > 0,
                                         X[4 * b + 2:4 * b + 3, :],
                                         lf1_ref[b:b + 1, :])
        lf2n_ref[b:b + 1, :] = jnp.where(lens_b > 0, o1_last,
                                         lf2_ref[b:b + 1, :])

        @pl.when(lens_b > 0)
        def _(b=b, r=r):
            pltpu.make_async_copy(
                yo_s.at[pl.ds(2 * b, 1), :],
                out_hbm.at[pl.ds(r, 1), :],
                out_sems.at[2 * b],
            ).start()

        @pl.when(lens_b > 1)
        def _(b=b, r=r):
            pltpu.make_async_copy(
                yo_s.at[pl.ds(2 * b + 1, 1), :],
                out_hbm.at[pl.ds(r + 1, 1), :],
                out_sems.at[2 * b + 1],
            ).start()

    for b in range(_B):
        lens_b = s_ref[b + 1] - s_ref[b]

        @pl.when(lens_b > 0)
        def _(b=b):
            pltpu.make_async_copy(
                yo_s.at[pl.ds(2 * b, 1), :],
                out_hbm.at[pl.ds(0, 1), :],
                out_sems.at[2 * b],
            ).wait()

        @pl.when(lens_b > 1)
        def _(b=b):
            pltpu.make_async_copy(
                yo_s.at[pl.ds(2 * b + 1, 1), :],
                out_hbm.at[pl.ds(0, 1), :],
                out_sems.at[2 * b + 1],
            ).wait()


@jax.jit
def kernel(inputs, lf1_cache, lf2_cache, conv1_weight, conv2_weight,
           conv1_bias, conv2_bias, ln_weight, cu_seqlens):
    lf1 = lf1_cache.reshape(_B, _D)
    lf2 = lf2_cache.reshape(_B, _H)
    b1 = conv1_bias.reshape(1, _H)
    b2 = conv2_bias.reshape(1, _D)
    ln = ln_weight.reshape(1, _D)
    n_tiles = _TOTAL // _T

    dense_out = pl.pallas_call(
        _dense_kernel,
        grid=(n_tiles,),
        in_specs=[
            pl.BlockSpec((_T, _D), lambda i: (i, 0)),
            pl.BlockSpec((_D, _D), lambda i: (0, 0)),
            pl.BlockSpec((_H, 2 * _D), lambda i: (0, 0)),
            pl.BlockSpec((1, _H), lambda i: (0, 0)),
            pl.BlockSpec((1, _D), lambda i: (0, 0)),
            pl.BlockSpec((1, _D), lambda i: (0, 0)),
        ],
        out_specs=pl.BlockSpec((_T, _D), lambda i: (i, 0)),
        out_shape=jax.ShapeDtypeStruct((_TOTAL, _D), jnp.float32),
        scratch_shapes=[
            pltpu.VMEM((1, _H), jnp.float32),
            pltpu.VMEM((1, _D), jnp.float32),
        ],
        compiler_params=pltpu.CompilerParams(
            dimension_semantics=("arbitrary",)),
    )(inputs, conv1_weight, conv2_weight, b1, b2, ln)

    grid_spec = pltpu.PrefetchScalarGridSpec(
        num_scalar_prefetch=1,
        grid=(1,),
        in_specs=[
            pl.BlockSpec(memory_space=pl.ANY),
            pl.BlockSpec(memory_space=pl.ANY),
            pl.BlockSpec((_B, _D), lambda i, s: (0, 0)),
            pl.BlockSpec((_B, _H), lambda i, s: (0, 0)),
            pl.BlockSpec((_D, _D), lambda i, s: (0, 0)),
            pl.BlockSpec((_H, 2 * _D), lambda i, s: (0, 0)),
            pl.BlockSpec((1, _H), lambda i, s: (0, 0)),
            pl.BlockSpec((1, _D), lambda i, s: (0, 0)),
            pl.BlockSpec((1, _D), lambda i, s: (0, 0)),
        ],
        out_specs=[
            pl.BlockSpec(memory_space=pl.ANY),
            pl.BlockSpec((_B, _D), lambda i, s: (0, 0)),
            pl.BlockSpec((_B, _H), lambda i, s: (0, 0)),
        ],
        scratch_shapes=[
            pltpu.VMEM((4 * _B, _D), jnp.float32),
            pltpu.VMEM((2 * _B, _D), jnp.float32),
            pltpu.SemaphoreType.DMA((4 * _B,)),
            pltpu.SemaphoreType.DMA((2 * _B,)),
        ],
    )

    out, lf1n, lf2n = pl.pallas_call(
        _fixup_kernel,
        grid_spec=grid_spec,
        out_shape=[
            jax.ShapeDtypeStruct((_TOTAL, _D), jnp.float32),
            jax.ShapeDtypeStruct((_B, _D), jnp.float32),
            jax.ShapeDtypeStruct((_B, _H), jnp.float32),
        ],
        input_output_aliases={2: 0},
    )(cu_seqlens, inputs, dense_out, lf1, lf2, conv1_weight, conv2_weight,
      b1, b2, ln)

    return out, lf1n.reshape(_B, 1, _D), lf2n.reshape(_B, 1, _H)


# speedup vs baseline: 1.1897x; 1.1017x over previous
import jax
import jax.numpy as jnp
from jax.experimental import pallas as pl
from jax.experimental.pallas import tpu as pltpu

_B = 8
_TOTAL = 8192
_D = 512
_H = _D // 2
_EPS = 1e-6
_T = 2048
_NC = 4
_C = _T // _NC


def _rms(y, ln):
    var = jnp.mean(y * y, axis=-1, keepdims=True)
    return y * jax.lax.rsqrt(var + _EPS) * ln


def _chunk(x, ca1, ca2, w1_ref, w2_ref, b1, b2, ln):
    c1 = jnp.dot(x, w1_ref[:], preferred_element_type=jnp.float32)
    c1h = c1[:, :_H]
    o1 = (jnp.concatenate([ca1, c1h[:_C - 1, :]], axis=0)
          + c1[:, _H:] + b1)
    c2 = jnp.dot(o1, w2_ref[:], preferred_element_type=jnp.float32)
    c2d = c2[:, :_D]
    y = (jnp.concatenate([ca2, c2d[:_C - 1, :]], axis=0)
         + c2[:, _D:] + b2 + x)
    return _rms(y, ln), c1h[_C - 1:_C, :], c2d[_C - 1:_C, :]


def _dense_kernel(x_ref, w1_ref, w2_ref, b1_ref, b2_ref, ln_ref, out_ref,
                  carry1, carry2):
    i = pl.program_id(0)

    @pl.when(i == 0)
    def _init():
        carry1[:] = jnp.zeros_like(carry1)
        carry2[:] = jnp.zeros_like(carry2)

    x = x_ref[:]
    ca1 = carry1[:]
    ca2 = carry2[:]
    for k in range(_NC):
        xk = x[k * _C:(k + 1) * _C, :]
        outk, ca1, ca2 = _chunk(xk, ca1, ca2, w1_ref, w2_ref,
                                b1_ref[:], b2_ref[:], ln_ref[:])
        out_ref[k * _C:(k + 1) * _C, :] = outk
    carry1[:] = ca1
    carry2[:] = ca2


def _fixup_kernel(s_ref, x_hbm, out_in_hbm, lf1_ref, lf2_ref, w1_ref,
                  w2_ref, b1_ref, b2_ref, ln_ref, out_hbm, lf1n_ref,
                  lf2n_ref, xr_s, yo_s, in_sems, out_sems):
    f32 = jnp.float32
    del out_in_hbm

    def clamp(v):
        return jnp.minimum(jnp.maximum(v, 0), _TOTAL - 1)

    for b in range(_B):
        r = s_ref[b]
        last = s_ref[b + 1] - 1
        for j, idx in enumerate((r, r + 1, last, last - 1)):
            pltpu.make_async_copy(
                x_hbm.at[pl.ds(clamp(idx), 1), :],
                xr_s.at[pl.ds(4 * b + j, 1), :],
                in_sems.at[4 * b + j],
            ).start()
    for b in range(_B):
        for j in range(4):
            pltpu.make_async_copy(
                x_hbm.at[pl.ds(0, 1), :],
                xr_s.at[pl.ds(4 * b + j, 1), :],
                in_sems.at[4 * b + j],
            ).wait()

    X = xr_s[:]
    C1 = jnp.dot(X, w1_ref[:], preferred_element_type=f32)
    cproj1 = jnp.dot(lf1_ref[:], w1_ref[:, :_H],
                     preferred_element_type=f32)
    cproj2 = jnp.dot(lf2_ref[:], w2_ref[:, :_D],
                     preferred_element_type=f32)

    b1 = b1_ref[:]
    b2 = b2_ref[:]
    ln = ln_ref[:]

    o1_rows = []
    for b in range(_B):
        o1_start = cproj1[b:b + 1, :] + C1[4 * b:4 * b + 1, _H:] + b1
        o1_succ = (C1[4 * b:4 * b + 1, :_H]
                   + C1[4 * b + 1:4 * b + 2, _H:] + b1)
        o1_rows.append(o1_start)
        o1_rows.append(o1_succ)
    O1 = jnp.concatenate(o1_rows, axis=0)
    C2 = jnp.dot(O1, w2_ref[:], preferred_element_type=f32)

    for b in range(_B):
        r = s_ref[b]
        lens_b = s_ref[b + 1] - s_ref[b]
        xr = X[4 * b:4 * b + 1, :]
        xr1 = X[4 * b + 1:4 * b + 2, :]
        y_r = (cproj2[b:b + 1, :] + C2[2 * b:2 * b + 1, _D:] + b2 + xr)
        y_r1 = (C2[2 * b:2 * b + 1, :_D]
                + C2[2 * b + 1:2 * b + 2, _D:] + b2 + xr1)
        yo_s[2 * b:2 * b + 1, :] = _rms(y_r, ln)
        yo_s[2 * b + 1:2 * b + 2, :] = _rms(y_r1, ln)

        o1_start = O1[2 * b:2 * b + 1, :]
        o1_last_dense = (C1[4 * b + 3:4 * b + 4, :_H]
                         + C1[4 * b + 2:4 * b + 3, _H:] + b1)
        o1_last = jnp.where(lens_b == 1, o1_start, o1_last_dense)
        lf1n_ref[b:b + 1, :] = jnp.where(lens_b > 0,
                                         X[4 * b + 2:4 * b + 3, :],
                                         lf1_ref[b:b + 1, :])
        lf2n_ref[b:b + 1, :] = jnp.where(lens_b > 0, o1_last,
                                         lf2_ref[b:b + 1, :])

        @pl.when(lens_b > 0)
        def _(b=b, r=r):
            pltpu.make_async_copy(
                yo_s.at[pl.ds(2 * b, 1), :],
                out_hbm.at[pl.ds(r, 1), :],
                out_sems.at[2 * b],
            ).start()

        @pl.when(lens_b > 1)
        def _(b=b, r=r):
            pltpu.make_async_copy(
                yo_s.at[pl.ds(2 * b + 1, 1), :],
                out_hbm.at[pl.ds(r + 1, 1), :],
                out_sems.at[2 * b + 1],
            ).start()

    for b in range(_B):
        lens_b = s_ref[b + 1] - s_ref[b]

        @pl.when(lens_b > 0)
        def _(b=b):
            pltpu.make_async_copy(
                yo_s.at[pl.ds(2 * b, 1), :],
                out_hbm.at[pl.ds(0, 1), :],
                out_sems.at[2 * b],
            ).wait()

        @pl.when(lens_b > 1)
        def _(b=b):
            pltpu.make_async_copy(
                yo_s.at[pl.ds(2 * b + 1, 1), :],
                out_hbm.at[pl.ds(0, 1), :],
                out_sems.at[2 * b + 1],
            ).wait()


@jax.jit
def kernel(inputs, lf1_cache, lf2_cache, conv1_weight, conv2_weight,
           conv1_bias, conv2_bias, ln_weight, cu_seqlens):
    lf1 = lf1_cache.reshape(_B, _D)
    lf2 = lf2_cache.reshape(_B, _H)
    b1 = conv1_bias.reshape(1, _H)
    b2 = conv2_bias.reshape(1, _D)
    ln = ln_weight.reshape(1, _D)
    n_tiles = _TOTAL // _T

    dense_out = pl.pallas_call(
        _dense_kernel,
        grid=(n_tiles,),
        in_specs=[
            pl.BlockSpec((_T, _D), lambda i: (i, 0)),
            pl.BlockSpec((_D, _D), lambda i: (0, 0)),
            pl.BlockSpec((_H, 2 * _D), lambda i: (0, 0)),
            pl.BlockSpec((1, _H), lambda i: (0, 0)),
            pl.BlockSpec((1, _D), lambda i: (0, 0)),
            pl.BlockSpec((1, _D), lambda i: (0, 0)),
        ],
        out_specs=pl.BlockSpec((_T, _D), lambda i: (i, 0)),
        out_shape=jax.ShapeDtypeStruct((_TOTAL, _D), jnp.float32),
        scratch_shapes=[
            pltpu.VMEM((1, _H), jnp.float32),
            pltpu.VMEM((1, _D), jnp.float32),
        ],
        compiler_params=pltpu.CompilerParams(
            dimension_semantics=("arbitrary",)),
    )(inputs, conv1_weight, conv2_weight, b1, b2, ln)

    grid_spec = pltpu.PrefetchScalarGridSpec(
        num_scalar_prefetch=1,
        grid=(1,),
        in_specs=[
            pl.BlockSpec(memory_space=pl.ANY),
            pl.BlockSpec(memory_space=pl.ANY),
            pl.BlockSpec((_B, _D), lambda i, s: (0, 0)),
            pl.BlockSpec((_B, _H), lambda i, s: (0, 0)),
            pl.BlockSpec((_D, _D), lambda i, s: (0, 0)),
            pl.BlockSpec((_H, 2 * _D), lambda i, s: (0, 0)),
            pl.BlockSpec((1, _H), lambda i, s: (0, 0)),
            pl.BlockSpec((1, _D), lambda i, s: (0, 0)),
            pl.BlockSpec((1, _D), lambda i, s: (0, 0)),
        ],
        out_specs=[
            pl.BlockSpec(memory_space=pl.ANY),
            pl.BlockSpec((_B, _D), lambda i, s: (0, 0)),
            pl.BlockSpec((_B, _H), lambda i, s: (0, 0)),
        ],
        scratch_shapes=[
            pltpu.VMEM((4 * _B, _D), jnp.float32),
            pltpu.VMEM((2 * _B, _D), jnp.float32),
            pltpu.SemaphoreType.DMA((4 * _B,)),
            pltpu.SemaphoreType.DMA((2 * _B,)),
        ],
    )

    out, lf1n, lf2n = pl.pallas_call(
        _fixup_kernel,
        grid_spec=grid_spec,
        out_shape=[
            jax.ShapeDtypeStruct((_TOTAL, _D), jnp.float32),
            jax.ShapeDtypeStruct((_B, _D), jnp.float32),
            jax.ShapeDtypeStruct((_B, _H), jnp.float32),
        ],
        input_output_aliases={2: 0},
    )(cu_seqlens, inputs, dense_out, lf1, lf2, conv1_weight, conv2_weight,
      b1, b2, ln)

    return out, lf1n.reshape(_B, 1, _D), lf2n.reshape(_B, 1, _H)
